# Initial kernel scaffold; baseline (speedup 1.0000x reference)
import jax
import jax.numpy as jnp
from jax.experimental import pallas as pl


def kernel(inputs, embedding_weight):
    raise NotImplementedError("write your pallas kernel here")


# speedup vs baseline: 1.0874x; 1.0874x over previous
import functools

import jax
import jax.numpy as jnp
from jax import lax
from jax.experimental import pallas as pl
from jax.experimental.pallas import tpu as pltpu

M = 4096
K = 32
N = 8192
R = 256
NB = M // R
COMMITMENT_COST = 0.25


def _vq_body(x_ref, xsq_ref, wsq_ref, w_ref, enc_ref, q_ref, idx_ref,
             loss_ref, perp_ref, counts_ref, acc_ref):
    step = pl.program_id(0)

    @pl.when(step == 0)
    def _init():
        counts_ref[...] = jnp.zeros_like(counts_ref)
        acc_ref[0] = 0.0

    x = x_ref[...]
    w = w_ref[...]
    xw = lax.dot_general(x, w, (((1,), (1,)), ((), ())),
                         preferred_element_type=jnp.float32)
    d = (xsq_ref[...] + wsq_ref[...]) - 2.0 * xw
    mval = jnp.min(d, axis=1, keepdims=True)
    iota_m = lax.broadcasted_iota(jnp.int32, (R, N), 1)
    idx = jnp.min(jnp.where(d == mval, iota_m, N), axis=1).astype(jnp.int32)

    iota = lax.broadcasted_iota(jnp.int32, (R, N), 1)
    enc = (iota == idx[:, None]).astype(jnp.float32)
    enc_ref[...] = enc
    idx_ref[0, 0, :] = idx

    q = lax.dot_general(enc, w, (((1,), (0,)), ((), ())),
                        preferred_element_type=jnp.float32)
    q_ref[...] = x + (q - x)

    counts_ref[...] += jnp.sum(enc, axis=0)[None, :]
    acc_ref[0] += jnp.sum((q - x) ** 2)

    @pl.when(step == NB - 1)
    def _fini():
        loss_ref[0, 0] = COMMITMENT_COST * acc_ref[0] / (M * K)
        p = counts_ref[...] * (1.0 / M)
        perp_ref[0, 0] = jnp.exp(-jnp.sum(p * jnp.log(p + 1e-10)))


@functools.partial(jax.jit, static_argnames=("interpret",))
def _vq_call(x_flat, xsq, wsq, embedding_weight, interpret=False):
    out_shapes = (
        jax.ShapeDtypeStruct((M, N), jnp.float32),
        jax.ShapeDtypeStruct((M, K), jnp.float32),
        jax.ShapeDtypeStruct((NB, 1, R), jnp.int32),
        jax.ShapeDtypeStruct((1, 1), jnp.float32),
        jax.ShapeDtypeStruct((1, 1), jnp.float32),
    )
    out_specs = (
        pl.BlockSpec((R, N), lambda i: (i, 0)),
        pl.BlockSpec((R, K), lambda i: (i, 0)),
        pl.BlockSpec((1, 1, R), lambda i: (i, 0, 0)),
        pl.BlockSpec(memory_space=pltpu.SMEM),
        pl.BlockSpec(memory_space=pltpu.SMEM),
    )
    in_specs = [
        pl.BlockSpec((R, K), lambda i: (i, 0)),
        pl.BlockSpec((R, 1), lambda i: (i, 0)),
        pl.BlockSpec((1, N), lambda i: (0, 0)),
        pl.BlockSpec((N, K), lambda i: (0, 0)),
    ]
    return pl.pallas_call(
        _vq_body,
        grid=(NB,),
        in_specs=in_specs,
        out_specs=out_specs,
        out_shape=out_shapes,
        scratch_shapes=[
            pltpu.VMEM((1, N), jnp.float32),
            pltpu.SMEM((1,), jnp.float32),
        ],
        interpret=interpret,
    )(x_flat, xsq, wsq, embedding_weight)


def kernel(inputs, embedding_weight, interpret=False):
    x = jnp.transpose(inputs, (0, 2, 3, 1))
    x_flat = x.reshape(M, K)
    xsq = jnp.sum(x_flat ** 2, axis=1, keepdims=True)
    wsq = jnp.sum(embedding_weight ** 2, axis=1).reshape(1, N)
    enc, q, idx, loss, perp = _vq_call(x_flat, xsq, wsq, embedding_weight,
                                       interpret=interpret)
    quantized_out = jnp.transpose(q.reshape(x.shape), (0, 3, 1, 2))
    return (quantized_out,
            loss.reshape(()),
            perp.reshape(()),
            idx.reshape(M, 1),
            enc)


# speedup vs baseline: 1.1372x; 1.0458x over previous
import functools

import jax
import jax.numpy as jnp
from jax import lax
from jax.experimental import pallas as pl
from jax.experimental.pallas import tpu as pltpu

M = 4096
K = 32
N = 8192
R = 256
NB = M // R
COMMITMENT_COST = 0.25


def _vq_body(x_ref, xm2_ref, xsq_ref, wsq_ref, w_ref, enc_ref, q_ref, idx_ref,
             loss_ref, perp_ref, counts_ref, acc_ref):
    step = pl.program_id(0)

    @pl.when(step == 0)
    def _init():
        counts_ref[...] = jnp.zeros_like(counts_ref)
        acc_ref[0] = 0.0

    x = x_ref[...]
    xm2 = xm2_ref[...]
    w = w_ref[...]
    xwm2 = lax.dot_general(xm2, w, (((1,), (1,)), ((), ())),
                           preferred_element_type=jnp.float32)
    d = (xsq_ref[...] + wsq_ref[...]) + xwm2
    mval = jnp.min(d, axis=1, keepdims=True)
    iota = lax.broadcasted_iota(jnp.int32, (R, N), 1)
    idx = jnp.min(jnp.where(d == mval, iota, N), axis=1)
    idx_ref[0, 0, :] = idx

    enc = (iota == idx[:, None]).astype(jnp.float32)
    enc_ref[...] = enc

    q = lax.dot_general(enc, w, (((1,), (0,)), ((), ())),
                        preferred_element_type=jnp.float32)
    q_ref[...] = x + (q - x)

    ones_r = jnp.ones((1, R), jnp.float32)
    counts_ref[...] += lax.dot_general(ones_r, enc, (((1,), (0,)), ((), ())),
                                       preferred_element_type=jnp.float32)
    acc_ref[0] += jnp.sum((q - x) ** 2)

    @pl.when(step == NB - 1)
    def _fini():
        loss_ref[0, 0] = COMMITMENT_COST * acc_ref[0] / (M * K)
        p = counts_ref[...] * (1.0 / M)
        perp_ref[0, 0] = jnp.exp(-jnp.sum(p * jnp.log(p + 1e-10)))


@functools.partial(jax.jit, static_argnames=("interpret",))
def _vq_call(x_flat, xm2, xsq, wsq, embedding_weight, interpret=False):
    out_shapes = (
        jax.ShapeDtypeStruct((M, N), jnp.float32),
        jax.ShapeDtypeStruct((M, K), jnp.float32),
        jax.ShapeDtypeStruct((NB, 1, R), jnp.int32),
        jax.ShapeDtypeStruct((1, 1), jnp.float32),
        jax.ShapeDtypeStruct((1, 1), jnp.float32),
    )
    out_specs = (
        pl.BlockSpec((R, N), lambda i: (i, 0)),
        pl.BlockSpec((R, K), lambda i: (i, 0)),
        pl.BlockSpec((1, 1, R), lambda i: (i, 0, 0)),
        pl.BlockSpec(memory_space=pltpu.SMEM),
        pl.BlockSpec(memory_space=pltpu.SMEM),
    )
    in_specs = [
        pl.BlockSpec((R, K), lambda i: (i, 0)),
        pl.BlockSpec((R, K), lambda i: (i, 0)),
        pl.BlockSpec((R, 1), lambda i: (i, 0)),
        pl.BlockSpec((1, N), lambda i: (0, 0)),
        pl.BlockSpec((N, K), lambda i: (0, 0)),
    ]
    return pl.pallas_call(
        _vq_body,
        grid=(NB,),
        in_specs=in_specs,
        out_specs=out_specs,
        out_shape=out_shapes,
        scratch_shapes=[
            pltpu.VMEM((1, N), jnp.float32),
            pltpu.SMEM((1,), jnp.float32),
        ],
        interpret=interpret,
    )(x_flat, xm2, xsq, wsq, embedding_weight)


def kernel(inputs, embedding_weight, interpret=False):
    x = jnp.transpose(inputs, (0, 2, 3, 1))
    x_flat = x.reshape(M, K)
    xm2 = x_flat * (-2.0)
    xsq = jnp.sum(x_flat ** 2, axis=1, keepdims=True)
    wsq = jnp.sum(embedding_weight ** 2, axis=1).reshape(1, N)
    enc, q, idx, loss, perp = _vq_call(x_flat, xm2, xsq, wsq, embedding_weight,
                                       interpret=interpret)
    quantized_out = jnp.transpose(q.reshape(x.shape), (0, 3, 1, 2))
    return (quantized_out,
            loss.reshape(()),
            perp.reshape(()),
            idx.reshape(M, 1),
            enc)
